# Initial kernel scaffold; baseline (speedup 1.0000x reference)
import jax
import jax.numpy as jnp
from jax.experimental import pallas as pl


def kernel(x, table):
    raise NotImplementedError("write your pallas kernel here")


# speedup vs baseline: 7.3147x; 7.3147x over previous
import functools

import jax

jax.config.update("jax_enable_x64", True)
import jax.numpy as jnp
from jax import lax
from jax.experimental import pallas as pl
from jax.experimental.pallas import tpu as pltpu
from jax.experimental.pallas import tpu_sc as plsc

NC = 2
NS = 16
L = 16
NW = NC * NS

N_POINTS = 4194304
TABLE_SIZE = 65536
BW = N_POINTS // NW
CHUNK = 8192
NCHUNK = BW // CHUNK
VPC = CHUNK // L

PRIME = jnp.int32(-1640531535)
HASH_MASK = jnp.int32(TABLE_SIZE - 1)


def _sc_body(x_hbm, table_hbm, out_hbm,
             table_v, xbuf0, xbuf1, obuf0, obuf1,
             sem_t, sem_in0, sem_in1, sem_out0, sem_out1):
    wid = lax.axis_index("s") * NC + lax.axis_index("c")
    base = wid * BW

    table_cp = pltpu.async_copy(table_hbm, table_v, sem_t)

    xbufs = [xbuf0, xbuf1]
    obufs = [obuf0, obuf1]
    sems_in = [sem_in0, sem_in1]
    sems_out = [sem_out0, sem_out1]

    lane2 = lax.iota(jnp.int32, L) * 2

    in_copies = [None, None]
    out_copies = [None, None]
    in_copies[0] = pltpu.async_copy(
        x_hbm.at[pl.ds(2 * base, 2 * CHUNK)], xbufs[0], sems_in[0])
    table_cp.wait()

    for k in range(NCHUNK):
        cur = k % 2
        nxt = (k + 1) % 2
        if k + 1 < NCHUNK:
            in_copies[nxt] = pltpu.async_copy(
                x_hbm.at[pl.ds(2 * (base + (k + 1) * CHUNK), 2 * CHUNK)],
                xbufs[nxt], sems_in[nxt])
        in_copies[cur].wait()
        if k >= 2:
            out_copies[cur].wait()

        xb = xbufs[cur]
        ob = obufs[cur]

        def inner(i, carry, xb=xb, ob=ob):
            rows_x = i * (2 * L) + lane2
            xv = plsc.load_gather(xb, [rows_x])
            yv = plsc.load_gather(xb, [rows_x + 1])
            c0 = (xv * 0.5).astype(jnp.int32)
            c1 = (yv * 0.5).astype(jnp.int32)
            h = (c0 ^ (c1 * PRIME)) & HASH_MASK
            ob[pl.ds(i * L, L)] = plsc.load_gather(table_v, [h])
            return carry

        lax.fori_loop(jnp.int32(0), jnp.int32(VPC), inner, jnp.int32(0))

        out_copies[cur] = pltpu.async_copy(
            ob, out_hbm.at[pl.ds(base + k * CHUNK, CHUNK)], sems_out[cur])

    out_copies[NCHUNK % 2].wait()
    out_copies[(NCHUNK + 1) % 2].wait()


_sc_call = pl.kernel(
    _sc_body,
    out_type=jax.ShapeDtypeStruct((N_POINTS,), jnp.float32),
    mesh=plsc.VectorSubcoreMesh(core_axis_name="c", subcore_axis_name="s",
                                num_cores=NC, num_subcores=NS),
    compiler_params=pltpu.CompilerParams(needs_layout_passes=False),
    scratch_types=[
        pltpu.VMEM((TABLE_SIZE,), jnp.float32),
        pltpu.VMEM((2 * CHUNK,), jnp.float32),
        pltpu.VMEM((2 * CHUNK,), jnp.float32),
        pltpu.VMEM((CHUNK,), jnp.float32),
        pltpu.VMEM((CHUNK,), jnp.float32),
        pltpu.SemaphoreType.DMA,
        pltpu.SemaphoreType.DMA,
        pltpu.SemaphoreType.DMA,
        pltpu.SemaphoreType.DMA,
        pltpu.SemaphoreType.DMA,
    ],
)


def kernel(x, table):
    return _sc_call(x.reshape(2 * N_POINTS), table.reshape(TABLE_SIZE))


# speedup vs baseline: 7.4635x; 1.0203x over previous
import functools

import jax

jax.config.update("jax_enable_x64", True)
import jax.numpy as jnp
from jax import lax
from jax.experimental import pallas as pl
from jax.experimental.pallas import tpu as pltpu
from jax.experimental.pallas import tpu_sc as plsc

NC = 2
NS = 16
L = 16
NW = NC * NS

N_POINTS = 4194304
TABLE_SIZE = 65536
BW = N_POINTS // NW
CHUNK = 8192
NCHUNK = BW // CHUNK
VPC = CHUNK // L

PRIME = jnp.int32(-1640531535)
HASH_MASK = jnp.int32(TABLE_SIZE - 1)


def _sc_body(x_hbm, table_hbm, out_hbm,
             table_v, xbuf0, xbuf1, obuf0, obuf1,
             sem_t, sem_in0, sem_in1, sem_out0, sem_out1):
    wid = lax.axis_index("s") * NC + lax.axis_index("c")
    base = wid * BW

    table_cp = pltpu.async_copy(table_hbm, table_v, sem_t)

    xbufs = [xbuf0, xbuf1]
    obufs = [obuf0, obuf1]
    sems_in = [sem_in0, sem_in1]
    sems_out = [sem_out0, sem_out1]

    lane2 = lax.iota(jnp.int32, L) * 2

    in_copies = [None, None]
    out_copies = [None, None]
    in_copies[0] = pltpu.async_copy(
        x_hbm.at[pl.ds(2 * base, 2 * CHUNK)], xbufs[0], sems_in[0])
    table_cp.wait()

    for k in range(NCHUNK):
        cur = k % 2
        nxt = (k + 1) % 2
        if k + 1 < NCHUNK:
            in_copies[nxt] = pltpu.async_copy(
                x_hbm.at[pl.ds(2 * (base + (k + 1) * CHUNK), 2 * CHUNK)],
                xbufs[nxt], sems_in[nxt])
        in_copies[cur].wait()
        if k >= 2:
            out_copies[cur].wait()

        xb = xbufs[cur]
        ob = obufs[cur]

        @plsc.parallel_loop(jnp.int32(0), jnp.int32(VPC), jnp.int32(1),
                            unroll=8, carry=jnp.int32(0))
        def _inner(i, j, xb=xb, ob=ob):
            rows_x = j * (2 * L) + lane2
            xv = plsc.load_gather(xb, [rows_x])
            yv = plsc.load_gather(xb, [rows_x + 1])
            c0 = (xv * 0.5).astype(jnp.int32)
            c1 = (yv * 0.5).astype(jnp.int32)
            h = (c0 ^ (c1 * PRIME)) & HASH_MASK
            ob[pl.ds(j * L, L)] = plsc.load_gather(table_v, [h])
            return j + 1

        out_copies[cur] = pltpu.async_copy(
            ob, out_hbm.at[pl.ds(base + k * CHUNK, CHUNK)], sems_out[cur])

    out_copies[NCHUNK % 2].wait()
    out_copies[(NCHUNK + 1) % 2].wait()


_sc_call = pl.kernel(
    _sc_body,
    out_type=jax.ShapeDtypeStruct((N_POINTS,), jnp.float32),
    mesh=plsc.VectorSubcoreMesh(core_axis_name="c", subcore_axis_name="s",
                                num_cores=NC, num_subcores=NS),
    compiler_params=pltpu.CompilerParams(needs_layout_passes=False),
    scratch_types=[
        pltpu.VMEM((TABLE_SIZE,), jnp.float32),
        pltpu.VMEM((2 * CHUNK,), jnp.float32),
        pltpu.VMEM((2 * CHUNK,), jnp.float32),
        pltpu.VMEM((CHUNK,), jnp.float32),
        pltpu.VMEM((CHUNK,), jnp.float32),
        pltpu.SemaphoreType.DMA,
        pltpu.SemaphoreType.DMA,
        pltpu.SemaphoreType.DMA,
        pltpu.SemaphoreType.DMA,
        pltpu.SemaphoreType.DMA,
    ],
)


def kernel(x, table):
    return _sc_call(x.reshape(2 * N_POINTS), table.reshape(TABLE_SIZE))


# speedup vs baseline: 461.9433x; 61.8940x over previous
import functools

import jax

jax.config.update("jax_enable_x64", True)
import jax.numpy as jnp
from jax import lax
from jax.experimental import pallas as pl
from jax.experimental.pallas import tpu as pltpu
from jax.experimental.pallas import tpu_sc as plsc

NC = 2
NS = 16
L = 16
NW = NC * NS

N_POINTS = 4194304
TABLE_SIZE = 65536
BW = N_POINTS // NW
CHUNK = 8192
NCHUNK = BW // CHUNK
VPC = CHUNK // L

PRIME = jnp.int32(-1640531535)
HASH_MASK = jnp.int32(TABLE_SIZE - 1)


def _sc_body(xs_hbm, ys_hbm, table_hbm, out_hbm,
             table_v, xb0, xb1, yb0, yb1, ob0, ob1,
             sem_t, sem_x0, sem_x1, sem_y0, sem_y1, sem_o0, sem_o1):
    wid = lax.axis_index("s") * NC + lax.axis_index("c")
    base = wid * BW

    table_cp = pltpu.async_copy(table_hbm, table_v, sem_t)

    xbufs = [xb0, xb1]
    ybufs = [yb0, yb1]
    obufs = [ob0, ob1]
    sems_x = [sem_x0, sem_x1]
    sems_y = [sem_y0, sem_y1]
    sems_o = [sem_o0, sem_o1]

    x_copies = [None, None]
    y_copies = [None, None]
    o_copies = [None, None]
    x_copies[0] = pltpu.async_copy(
        xs_hbm.at[pl.ds(base, CHUNK)], xbufs[0], sems_x[0])
    y_copies[0] = pltpu.async_copy(
        ys_hbm.at[pl.ds(base, CHUNK)], ybufs[0], sems_y[0])
    table_cp.wait()

    for k in range(NCHUNK):
        cur = k % 2
        nxt = (k + 1) % 2
        if k + 1 < NCHUNK:
            nb = base + (k + 1) * CHUNK
            x_copies[nxt] = pltpu.async_copy(
                xs_hbm.at[pl.ds(nb, CHUNK)], xbufs[nxt], sems_x[nxt])
            y_copies[nxt] = pltpu.async_copy(
                ys_hbm.at[pl.ds(nb, CHUNK)], ybufs[nxt], sems_y[nxt])
        x_copies[cur].wait()
        y_copies[cur].wait()
        if k >= 2:
            o_copies[cur].wait()

        xb = xbufs[cur]
        yb = ybufs[cur]
        ob = obufs[cur]

        @plsc.parallel_loop(jnp.int32(0), jnp.int32(VPC), jnp.int32(1),
                            unroll=8, carry=jnp.int32(0))
        def _inner(i, j, xb=xb, yb=yb, ob=ob):
            sl = pl.ds(j * L, L)
            c0 = (xb[sl] * 0.5).astype(jnp.int32)
            c1 = (yb[sl] * 0.5).astype(jnp.int32)
            h = (c0 ^ (c1 * PRIME)) & HASH_MASK
            ob[sl] = plsc.load_gather(table_v, [h])
            return j + 1

        o_copies[cur] = pltpu.async_copy(
            ob, out_hbm.at[pl.ds(base + k * CHUNK, CHUNK)], sems_o[cur])

    o_copies[NCHUNK % 2].wait()
    o_copies[(NCHUNK + 1) % 2].wait()


_sc_call = pl.kernel(
    _sc_body,
    out_type=jax.ShapeDtypeStruct((N_POINTS,), jnp.float32),
    mesh=plsc.VectorSubcoreMesh(core_axis_name="c", subcore_axis_name="s",
                                num_cores=NC, num_subcores=NS),
    compiler_params=pltpu.CompilerParams(needs_layout_passes=False),
    scratch_types=[
        pltpu.VMEM((TABLE_SIZE,), jnp.float32),
        pltpu.VMEM((CHUNK,), jnp.float32),
        pltpu.VMEM((CHUNK,), jnp.float32),
        pltpu.VMEM((CHUNK,), jnp.float32),
        pltpu.VMEM((CHUNK,), jnp.float32),
        pltpu.VMEM((CHUNK,), jnp.float32),
        pltpu.VMEM((CHUNK,), jnp.float32),
        pltpu.SemaphoreType.DMA,
        pltpu.SemaphoreType.DMA,
        pltpu.SemaphoreType.DMA,
        pltpu.SemaphoreType.DMA,
        pltpu.SemaphoreType.DMA,
        pltpu.SemaphoreType.DMA,
        pltpu.SemaphoreType.DMA,
    ],
)


def kernel(x, table):
    return _sc_call(x[:, 0], x[:, 1], table.reshape(TABLE_SIZE))


# speedup vs baseline: 615.1865x; 1.3317x over previous
import functools

import jax

jax.config.update("jax_enable_x64", True)
import jax.numpy as jnp
from jax import lax
from jax.experimental import pallas as pl
from jax.experimental.pallas import tpu as pltpu
from jax.experimental.pallas import tpu_sc as plsc

NC = 2
NS = 16
L = 16
NW = NC * NS

N_POINTS = 4194304
TABLE_SIZE = 65536
BW = N_POINTS // NW
CHUNK = 8192
NCHUNK = BW // CHUNK
BPC = CHUNK // 128

PRIME = jnp.int32(-1640531535)
HASH_MASK = jnp.int32(TABLE_SIZE - 1)


def _sc_body(xz_hbm, table_hbm, out_hbm,
             table_v, xb0, xb1, ob0, ob1,
             sem_t, sem_x0, sem_x1, sem_o0, sem_o1):
    wid = lax.axis_index("s") * NC + lax.axis_index("c")
    base = wid * BW

    table_cp = pltpu.async_copy(table_hbm, table_v, sem_t)

    xbufs = [xb0, xb1]
    obufs = [ob0, ob1]
    sems_x = [sem_x0, sem_x1]
    sems_o = [sem_o0, sem_o1]

    x_copies = [None, None]
    o_copies = [None, None]
    x_copies[0] = pltpu.async_copy(
        xz_hbm.at[pl.ds(2 * base, 2 * CHUNK)], xbufs[0], sems_x[0])
    table_cp.wait()

    for k in range(NCHUNK):
        cur = k % 2
        nxt = (k + 1) % 2
        if k + 1 < NCHUNK:
            x_copies[nxt] = pltpu.async_copy(
                xz_hbm.at[pl.ds(2 * (base + (k + 1) * CHUNK), 2 * CHUNK)],
                xbufs[nxt], sems_x[nxt])
        x_copies[cur].wait()
        if k >= 2:
            o_copies[cur].wait()

        xb = xbufs[cur]
        ob = obufs[cur]

        @plsc.parallel_loop(jnp.int32(0), jnp.int32(BPC), jnp.int32(1),
                            unroll=2, carry=jnp.int32(0))
        def _inner(i, b, xb=xb, ob=ob):
            xoff = b * 256
            ooff = b * 128
            for g in range(8):
                xs = xb[pl.ds(xoff + g * L, L)]
                ys = xb[pl.ds(xoff + 128 + g * L, L)]
                c0 = (xs * 0.5).astype(jnp.int32)
                c1 = (ys * 0.5).astype(jnp.int32)
                h = (c0 ^ (c1 * PRIME)) & HASH_MASK
                ob[pl.ds(ooff + g * L, L)] = plsc.load_gather(table_v, [h])
            return b + 1

        o_copies[cur] = pltpu.async_copy(
            ob, out_hbm.at[pl.ds(base + k * CHUNK, CHUNK)], sems_o[cur])

    o_copies[NCHUNK % 2].wait()
    o_copies[(NCHUNK + 1) % 2].wait()


_sc_call = pl.kernel(
    _sc_body,
    out_type=jax.ShapeDtypeStruct((N_POINTS,), jnp.float32),
    mesh=plsc.VectorSubcoreMesh(core_axis_name="c", subcore_axis_name="s",
                                num_cores=NC, num_subcores=NS),
    compiler_params=pltpu.CompilerParams(needs_layout_passes=False),
    scratch_types=[
        pltpu.VMEM((TABLE_SIZE,), jnp.float32),
        pltpu.VMEM((2 * CHUNK,), jnp.float32),
        pltpu.VMEM((2 * CHUNK,), jnp.float32),
        pltpu.VMEM((CHUNK,), jnp.float32),
        pltpu.VMEM((CHUNK,), jnp.float32),
        pltpu.SemaphoreType.DMA,
        pltpu.SemaphoreType.DMA,
        pltpu.SemaphoreType.DMA,
        pltpu.SemaphoreType.DMA,
        pltpu.SemaphoreType.DMA,
    ],
)


def kernel(x, table):
    xz = x.reshape(32768, 128, 2).transpose(0, 2, 1).reshape(2 * N_POINTS)
    return _sc_call(xz, table.reshape(TABLE_SIZE))


# speedup vs baseline: 741.5453x; 1.2054x over previous
import functools

import jax

jax.config.update("jax_enable_x64", True)
import jax.numpy as jnp
from jax import lax
from jax.experimental import pallas as pl
from jax.experimental.pallas import tpu as pltpu
from jax.experimental.pallas import tpu_sc as plsc

NC = 2
NS = 16
L = 16
NW = NC * NS

N_POINTS = 4194304
TABLE_SIZE = 65536
BW = N_POINTS // NW
CHUNK = 8192
NCHUNK = BW // CHUNK
BPC = CHUNK // 128

PRIME = jnp.int32(-1640531535)
HASH_MASK = jnp.int32(TABLE_SIZE - 1)


def _sc_body(xz_hbm, table_hbm, out_hbm,
             table_v, xb0, xb1, ob0, ob1,
             sem_t, sem_x0, sem_x1, sem_o0, sem_o1):
    wid = lax.axis_index("s") * NC + lax.axis_index("c")
    base = wid * BW

    table_cp = pltpu.async_copy(table_hbm, table_v, sem_t)

    xbufs = [xb0, xb1]
    obufs = [ob0, ob1]
    sems_x = [sem_x0, sem_x1]
    sems_o = [sem_o0, sem_o1]

    pltpu.async_copy(
        xz_hbm.at[pl.ds(2 * base, 2 * CHUNK)], xbufs[0], sems_x[0])
    pltpu.async_copy(
        xz_hbm.at[pl.ds(2 * (base + CHUNK), 2 * CHUNK)], xbufs[1], sems_x[1])
    table_cp.wait()

    def _pair(t, carry):
        for half in range(2):
            xb = xbufs[half]
            ob = obufs[half]
            off = base + (2 * t + half) * CHUNK
            pltpu.make_async_copy(
                xz_hbm.at[pl.ds(2 * off, 2 * CHUNK)], xb,
                sems_x[half]).wait()

            @pl.when(t > 0)
            def _(ob=ob, off=off, half=half):
                pltpu.make_async_copy(
                    ob, out_hbm.at[pl.ds(off - 2 * CHUNK, CHUNK)],
                    sems_o[half]).wait()

            @plsc.parallel_loop(jnp.int32(0), jnp.int32(BPC), jnp.int32(1),
                                unroll=4, carry=jnp.int32(0))
            def _inner(i, b, xb=xb, ob=ob):
                xoff = b * 256
                ooff = b * 128
                for g in range(8):
                    xs = xb[pl.ds(xoff + g * L, L)]
                    ys = xb[pl.ds(xoff + 128 + g * L, L)]
                    c0 = (xs * 0.5).astype(jnp.int32)
                    c1 = (ys * 0.5).astype(jnp.int32)
                    h = (c0 ^ (c1 * PRIME)) & HASH_MASK
                    ob[pl.ds(ooff + g * L, L)] = plsc.load_gather(
                        table_v, [h])
                return b + 1

            pltpu.async_copy(
                ob, out_hbm.at[pl.ds(off, CHUNK)], sems_o[half])

            @pl.when(t < NCHUNK // 2 - 1)
            def _(xb=xb, off=off, half=half):
                pltpu.async_copy(
                    xz_hbm.at[pl.ds(2 * (off + 2 * CHUNK), 2 * CHUNK)], xb,
                    sems_x[half])
        return carry

    lax.fori_loop(jnp.int32(0), jnp.int32(NCHUNK // 2), _pair, jnp.int32(0))

    pltpu.make_async_copy(
        obufs[0], out_hbm.at[pl.ds(base + (NCHUNK - 2) * CHUNK, CHUNK)],
        sems_o[0]).wait()
    pltpu.make_async_copy(
        obufs[1], out_hbm.at[pl.ds(base + (NCHUNK - 1) * CHUNK, CHUNK)],
        sems_o[1]).wait()


_sc_call = pl.kernel(
    _sc_body,
    out_type=jax.ShapeDtypeStruct((N_POINTS,), jnp.float32),
    mesh=plsc.VectorSubcoreMesh(core_axis_name="c", subcore_axis_name="s",
                                num_cores=NC, num_subcores=NS),
    compiler_params=pltpu.CompilerParams(needs_layout_passes=False),
    scratch_types=[
        pltpu.VMEM((TABLE_SIZE,), jnp.float32),
        pltpu.VMEM((2 * CHUNK,), jnp.float32),
        pltpu.VMEM((2 * CHUNK,), jnp.float32),
        pltpu.VMEM((CHUNK,), jnp.float32),
        pltpu.VMEM((CHUNK,), jnp.float32),
        pltpu.SemaphoreType.DMA,
        pltpu.SemaphoreType.DMA,
        pltpu.SemaphoreType.DMA,
        pltpu.SemaphoreType.DMA,
        pltpu.SemaphoreType.DMA,
    ],
)


def kernel(x, table):
    xz = x.reshape(32768, 128, 2).transpose(0, 2, 1).reshape(2 * N_POINTS)
    return _sc_call(xz, table.reshape(TABLE_SIZE))
